# baseline (speedup 1.0000x reference)
import functools

import jax
import jax.numpy as jnp
from jax import lax
from jax.experimental import pallas as pl
from jax.experimental.pallas import tpu as pltpu
from jax.experimental.pallas import tpu_sc as plsc

_N = 10000
_D = 128
_E = 320000
_NC = 2
_NS = 16
_NW = _NC * _NS
_EPW = _E // _NW
_K = 40
_CPT = _EPW // _K
_RB = 9
_G = 5
_W = 3
_IB = 12
_IA = 8
_STRIPE = 640

_mesh = plsc.VectorSubcoreMesh(core_axis_name="c", subcore_axis_name="s")


@functools.partial(
    pl.kernel,
    out_type=jax.ShapeDtypeStruct((_NC, _N, _D), jnp.float32),
    mesh=_mesh,
    scratch_types=[
        pltpu.VMEM((_IB, _K), jnp.int32),
        pltpu.VMEM((_IB, _K), jnp.int32),
        pltpu.VMEM((_RB, _K, _D), jnp.float32),
        pltpu.VMEM_SHARED((_N, _D), jnp.float32),
        pltpu.SemaphoreType.DMA((_IB,)),
        pltpu.SemaphoreType.DMA((_RB,)),
        pltpu.SemaphoreType.DMA((_RB,)),
        pltpu.SemaphoreType.DMA,
    ],
)
def _segment_sum_sc(h_hbm, src_hbm, dst_hbm, out_hbm,
                    src_ring, dst_ring, rows_ring, acc_sh, isems, gsems,
                    ssems, zsem):
    c = lax.axis_index("c")
    s = lax.axis_index("s")
    wid = s * _NC + c
    row0 = pl.multiple_of(jnp.minimum(s * _STRIPE, _N - _STRIPE), 8)
    base = wid * _EPW

    zbuf = rows_ring.at[_RB - 1]

    def zvec(i, carry):
        for j in range(_D // 16):
            zbuf[i, pl.ds(j * 16, 16)] = jnp.zeros((16,), jnp.float32)
        return carry

    lax.fori_loop(0, _K, zvec, 0)
    for k in range(_STRIPE // _K):
        pltpu.async_copy(zbuf, acc_sh.at[pl.ds(row0 + k * _K, _K)], zsem)

    def idx_start(chunk):
        bi = chunk % _IB
        off = pl.multiple_of(base + chunk * _K, 8)
        pltpu.async_copy(src_hbm.at[pl.ds(off, _K)], src_ring.at[bi], isems.at[bi])
        pltpu.async_copy(dst_hbm.at[pl.ds(off, _K)], dst_ring.at[bi], isems.at[bi])

    def idx_wait(chunk):
        bi = chunk % _IB
        off = pl.multiple_of(base + chunk * _K, 8)
        pltpu.make_async_copy(src_hbm.at[pl.ds(off, _K)], src_ring.at[bi],
                              isems.at[bi]).wait()
        pltpu.make_async_copy(dst_hbm.at[pl.ds(off, _K)], dst_ring.at[bi],
                              isems.at[bi]).wait()

    def gather_start(chunk):
        b = chunk % _RB
        pltpu.async_copy(h_hbm.at[src_ring.at[chunk % _IB]], rows_ring.at[b],
                         gsems.at[b])

    def gather_wait(chunk):
        b = chunk % _RB
        pltpu.make_async_copy(h_hbm.at[src_ring.at[chunk % _IB]],
                              rows_ring.at[b], gsems.at[b]).wait()

    def scatter_start(chunk):
        b = chunk % _RB
        pltpu.async_copy(rows_ring.at[b], acc_sh.at[dst_ring.at[chunk % _IB]],
                         ssems.at[b], add=True)

    def scatter_wait(chunk):
        b = chunk % _RB
        pltpu.make_async_copy(rows_ring.at[b],
                              acc_sh.at[dst_ring.at[chunk % _IB]],
                              ssems.at[b]).wait()

    def step(chunk, *, wait_behind, prefetch):
        gather_wait(chunk)
        scatter_start(chunk)
        if wait_behind:
            scatter_wait(chunk - _W)
        if prefetch:
            ahead = chunk + _G
            if isinstance(chunk, int):
                if ahead < _CPT:
                    idx_wait(ahead)
                    gather_start(ahead)
                if chunk + _IA < _CPT:
                    idx_start(chunk + _IA)
            else:
                @pl.when(ahead < _CPT)
                def _():
                    idx_wait(ahead)
                    gather_start(ahead)

                @pl.when(chunk + _IA < _CPT)
                def _():
                    idx_start(chunk + _IA)

    for chunk in range(_IA):
        idx_start(chunk)
    for chunk in range(_G):
        idx_wait(chunk)
        gather_start(chunk)
    for k in range(_STRIPE // _K):
        pltpu.make_async_copy(zbuf, acc_sh.at[pl.ds(row0 + k * _K, _K)],
                              zsem).wait()
    plsc.subcore_barrier()

    for chunk in range(_W):
        step(chunk, wait_behind=False, prefetch=True)

    def body(chunk, carry):
        step(chunk, wait_behind=True, prefetch=True)
        return carry

    lax.fori_loop(_W, _CPT, body, 0)
    for chunk in range(_CPT - _W, _CPT):
        scatter_wait(chunk)
    plsc.subcore_barrier()
    pltpu.sync_copy(acc_sh.at[pl.ds(row0, _STRIPE)],
                    out_hbm.at[c].at[pl.ds(row0, _STRIPE)])


_BLK = 1000


def _affine_lrelu_body(acc_ref, w_ref, b_ref, o_ref):
    m = acc_ref[0] + acc_ref[1]
    y = jnp.dot(m, w_ref[...], preferred_element_type=jnp.float32) + b_ref[...]
    o_ref[...] = jnp.where(y >= 0.0, y, 0.1 * y)


def _affine_lrelu(acc, W, b):
    return pl.pallas_call(
        _affine_lrelu_body,
        grid=(_N // _BLK,),
        in_specs=[
            pl.BlockSpec((_NC, _BLK, _D), lambda i: (0, i, 0)),
            pl.BlockSpec((_D, _D), lambda i: (0, 0)),
            pl.BlockSpec((1, _D), lambda i: (0, 0)),
        ],
        out_specs=pl.BlockSpec((_BLK, _D), lambda i: (i, 0)),
        out_shape=jax.ShapeDtypeStruct((_N, _D), jnp.float32),
    )(acc, W, b.reshape(1, _D))


def _final_body(acc_ref, w2_ref, b2_ref, x_ref, h1_ref, h2_ref,
                wo_ref, bo_ref, o_ref):
    m = acc_ref[0] + acc_ref[1]
    y3 = jnp.dot(m, w2_ref[...], preferred_element_type=jnp.float32) + b2_ref[...]
    h3 = jnp.where(y3 >= 0.0, y3, 0.1 * y3)
    y = jnp.dot(x_ref[...], wo_ref[0], preferred_element_type=jnp.float32)
    y = y + jnp.dot(h1_ref[...], wo_ref[1], preferred_element_type=jnp.float32)
    y = y + jnp.dot(h2_ref[...], wo_ref[2], preferred_element_type=jnp.float32)
    y = y + jnp.dot(h3, wo_ref[3], preferred_element_type=jnp.float32)
    y = y + bo_ref[...]
    o_ref[...] = jnp.where(y >= 0.0, y, 0.1 * y)


def _final_proj(acc3, W2, b2, x, h1, h2, W_out, b_out):
    return pl.pallas_call(
        _final_body,
        grid=(_N // _BLK,),
        in_specs=[
            pl.BlockSpec((_NC, _BLK, _D), lambda i: (0, i, 0)),
            pl.BlockSpec((_D, _D), lambda i: (0, 0)),
            pl.BlockSpec((1, _D), lambda i: (0, 0)),
            pl.BlockSpec((_BLK, _D), lambda i: (i, 0)),
            pl.BlockSpec((_BLK, _D), lambda i: (i, 0)),
            pl.BlockSpec((_BLK, _D), lambda i: (i, 0)),
            pl.BlockSpec((4, _D, _D), lambda i: (0, 0, 0)),
            pl.BlockSpec((1, _D), lambda i: (0, 0)),
        ],
        out_specs=pl.BlockSpec((_BLK, _D), lambda i: (i, 0)),
        out_shape=jax.ShapeDtypeStruct((_N, _D), jnp.float32),
    )(acc3, W2, b2.reshape(1, _D), x, h1, h2,
      W_out.reshape(4, _D, _D), b_out.reshape(1, _D))


def kernel(x, edge_index, W0, b0, W1, b1, W2, b2, W_out, b_out):
    src = edge_index[0]
    dst = edge_index[1]
    acc1 = _segment_sum_sc(x, src, dst)
    h1 = _affine_lrelu(acc1, W0, b0)
    acc2 = _segment_sum_sc(h1, src, dst)
    h2 = _affine_lrelu(acc2, W1, b1)
    acc3 = _segment_sum_sc(h2, src, dst)
    return _final_proj(acc3, W2, b2, x, h1, h2, W_out, b_out)


# speedup vs baseline: 1.0116x; 1.0116x over previous
import functools

import jax
import jax.numpy as jnp
from jax import lax
from jax.experimental import pallas as pl
from jax.experimental.pallas import tpu as pltpu
from jax.experimental.pallas import tpu_sc as plsc

_N = 10000
_D = 128
_E = 320000
_NC = 2
_NS = 16
_NW = _NC * _NS
_EPW = _E // _NW
_K = 80
_CPT = _EPW // _K
_RB = 4
_IB = 6
_STRIPE = 640

_mesh = plsc.VectorSubcoreMesh(core_axis_name="c", subcore_axis_name="s")


@functools.partial(
    pl.kernel,
    out_type=jax.ShapeDtypeStruct((_NC, _N, _D), jnp.float32),
    mesh=_mesh,
    scratch_types=[
        pltpu.VMEM((_IB, _K), jnp.int32),
        pltpu.VMEM((_IB, _K), jnp.int32),
        pltpu.VMEM((_RB, _K, _D), jnp.float32),
        pltpu.VMEM_SHARED((_N, _D), jnp.float32),
        pltpu.SemaphoreType.DMA((_IB,)),
        pltpu.SemaphoreType.DMA((_RB,)),
        pltpu.SemaphoreType.DMA,
    ],
)
def _segment_sum_sc(h_hbm, src_hbm, dst_hbm, out_hbm,
                    src_ring, dst_ring, rows_ring, acc_sh, isems, gsems,
                    zsem):
    c = lax.axis_index("c")
    s = lax.axis_index("s")
    wid = s * _NC + c
    row0 = pl.multiple_of(jnp.minimum(s * _STRIPE, _N - _STRIPE), 8)
    base = wid * _EPW

    zbuf = rows_ring.at[_RB - 1]

    def zvec(i, carry):
        for j in range(_D // 16):
            zbuf[i, pl.ds(j * 16, 16)] = jnp.zeros((16,), jnp.float32)
        return carry

    lax.fori_loop(0, _K, zvec, 0)
    for k in range(_STRIPE // _K):
        pltpu.async_copy(zbuf, acc_sh.at[pl.ds(row0 + k * _K, _K)], zsem)

    def idx_start(chunk):
        bi = chunk % _IB
        off = pl.multiple_of(base + chunk * _K, 8)
        pltpu.async_copy(src_hbm.at[pl.ds(off, _K)], src_ring.at[bi], isems.at[bi])
        pltpu.async_copy(dst_hbm.at[pl.ds(off, _K)], dst_ring.at[bi], isems.at[bi])

    def idx_wait(chunk):
        bi = chunk % _IB
        off = pl.multiple_of(base + chunk * _K, 8)
        pltpu.make_async_copy(src_hbm.at[pl.ds(off, _K)], src_ring.at[bi],
                              isems.at[bi]).wait()
        pltpu.make_async_copy(dst_hbm.at[pl.ds(off, _K)], dst_ring.at[bi],
                              isems.at[bi]).wait()

    def gather_start(chunk):
        b = chunk % _RB
        pltpu.async_copy(h_hbm.at[src_ring.at[chunk % _IB]], rows_ring.at[b],
                         gsems.at[b])

    def gather_wait(chunk):
        b = chunk % _RB
        pltpu.make_async_copy(h_hbm.at[src_ring.at[chunk % _IB]],
                              rows_ring.at[b], gsems.at[b]).wait()

    def scatter_add(chunk):
        pltpu.sync_copy(rows_ring.at[chunk % _RB],
                        acc_sh.at[dst_ring.at[chunk % _IB]], add=True)

    for chunk in range(_IB):
        idx_start(chunk)
    for chunk in range(_RB - 1):
        idx_wait(chunk)
        gather_start(chunk)
    for k in range(_STRIPE // _K):
        pltpu.make_async_copy(zbuf, acc_sh.at[pl.ds(row0 + k * _K, _K)],
                              zsem).wait()
    plsc.subcore_barrier()

    def body(chunk, carry):
        gather_wait(chunk)
        scatter_add(chunk)
        ahead = chunk + _RB - 1

        @pl.when(ahead < _CPT)
        def _():
            idx_wait(ahead)
            gather_start(ahead)

        @pl.when(chunk + _IB < _CPT)
        def _():
            idx_start(chunk + _IB)

        return carry

    lax.fori_loop(0, _CPT, body, 0)
    plsc.subcore_barrier()
    pltpu.sync_copy(acc_sh.at[pl.ds(row0, _STRIPE)],
                    out_hbm.at[c].at[pl.ds(row0, _STRIPE)])


_BLK = 1000


def _affine_lrelu_body(acc_ref, w_ref, b_ref, o_ref):
    m = acc_ref[0] + acc_ref[1]
    y = jnp.dot(m, w_ref[...], preferred_element_type=jnp.float32) + b_ref[...]
    o_ref[...] = jnp.where(y >= 0.0, y, 0.1 * y)


def _affine_lrelu(acc, W, b):
    return pl.pallas_call(
        _affine_lrelu_body,
        grid=(_N // _BLK,),
        in_specs=[
            pl.BlockSpec((_NC, _BLK, _D), lambda i: (0, i, 0)),
            pl.BlockSpec((_D, _D), lambda i: (0, 0)),
            pl.BlockSpec((1, _D), lambda i: (0, 0)),
        ],
        out_specs=pl.BlockSpec((_BLK, _D), lambda i: (i, 0)),
        out_shape=jax.ShapeDtypeStruct((_N, _D), jnp.float32),
    )(acc, W, b.reshape(1, _D))


def _partial_body(x_ref, h1_ref, h2_ref, wo_ref, bo_ref, o_ref):
    y = jnp.dot(x_ref[...], wo_ref[0], preferred_element_type=jnp.float32)
    y = y + jnp.dot(h1_ref[...], wo_ref[1], preferred_element_type=jnp.float32)
    y = y + jnp.dot(h2_ref[...], wo_ref[2], preferred_element_type=jnp.float32)
    o_ref[...] = y + bo_ref[...]


def _partial_proj(x, h1, h2, W_out3, b_out):
    return pl.pallas_call(
        _partial_body,
        grid=(_N // _BLK,),
        in_specs=[
            pl.BlockSpec((_BLK, _D), lambda i: (i, 0)),
            pl.BlockSpec((_BLK, _D), lambda i: (i, 0)),
            pl.BlockSpec((_BLK, _D), lambda i: (i, 0)),
            pl.BlockSpec((3, _D, _D), lambda i: (0, 0, 0)),
            pl.BlockSpec((1, _D), lambda i: (0, 0)),
        ],
        out_specs=pl.BlockSpec((_BLK, _D), lambda i: (i, 0)),
        out_shape=jax.ShapeDtypeStruct((_N, _D), jnp.float32),
    )(x, h1, h2, W_out3, b_out.reshape(1, _D))


def _final_body(acc_ref, w2_ref, b2_ref, part_ref, wo3_ref, o_ref):
    m = acc_ref[0] + acc_ref[1]
    y3 = jnp.dot(m, w2_ref[...], preferred_element_type=jnp.float32) + b2_ref[...]
    h3 = jnp.where(y3 >= 0.0, y3, 0.1 * y3)
    y = part_ref[...] + jnp.dot(h3, wo3_ref[...],
                                preferred_element_type=jnp.float32)
    o_ref[...] = jnp.where(y >= 0.0, y, 0.1 * y)


def _final_proj(acc3, W2, b2, part, W_out_h3):
    return pl.pallas_call(
        _final_body,
        grid=(_N // _BLK,),
        in_specs=[
            pl.BlockSpec((_NC, _BLK, _D), lambda i: (0, i, 0)),
            pl.BlockSpec((_D, _D), lambda i: (0, 0)),
            pl.BlockSpec((1, _D), lambda i: (0, 0)),
            pl.BlockSpec((_BLK, _D), lambda i: (i, 0)),
            pl.BlockSpec((_D, _D), lambda i: (0, 0)),
        ],
        out_specs=pl.BlockSpec((_BLK, _D), lambda i: (i, 0)),
        out_shape=jax.ShapeDtypeStruct((_N, _D), jnp.float32),
    )(acc3, W2, b2.reshape(1, _D), part, W_out_h3)


def kernel(x, edge_index, W0, b0, W1, b1, W2, b2, W_out, b_out):
    src = edge_index[0]
    dst = edge_index[1]
    Wo = W_out.reshape(4, _D, _D)
    acc1 = _segment_sum_sc(x, src, dst)
    h1 = _affine_lrelu(acc1, W0, b0)
    acc2 = _segment_sum_sc(h1, src, dst)
    h2 = _affine_lrelu(acc2, W1, b1)
    acc3 = _segment_sum_sc(h2, src, dst)
    part = _partial_proj(x, h1, h2, Wo[0:3], b_out)
    return _final_proj(acc3, W2, b2, part, Wo[3])


# speedup vs baseline: 1.0421x; 1.0301x over previous
import functools

import jax
import jax.numpy as jnp
from jax import lax
from jax.experimental import pallas as pl
from jax.experimental.pallas import tpu as pltpu
from jax.experimental.pallas import tpu_sc as plsc

_N = 10000
_D = 128
_E = 320000
_NC = 2
_NS = 16
_NW = _NC * _NS
_EPW = _E // _NW
_K = 80
_CPT = _EPW // _K
_RB = 4
_IB = 6
_STRIPE = 640

_mesh = plsc.VectorSubcoreMesh(core_axis_name="c", subcore_axis_name="s")


@functools.partial(
    pl.kernel,
    out_type=jax.ShapeDtypeStruct((_NC, _N, _D), jnp.float32),
    mesh=_mesh,
    scratch_types=[
        pltpu.VMEM((_IB, _K), jnp.int32),
        pltpu.VMEM((_IB, _K), jnp.int32),
        pltpu.VMEM((_RB, _K, _D), jnp.float32),
        pltpu.VMEM_SHARED((_N, _D), jnp.float32),
        pltpu.SemaphoreType.DMA((_IB,)),
        pltpu.SemaphoreType.DMA((_RB,)),
        pltpu.SemaphoreType.DMA,
    ],
)
def _segment_sum_sc(h_hbm, edges_hbm, out_hbm,
                    src_ring, dst_ring, rows_ring, acc_sh, isems, gsems,
                    zsem):
    c = lax.axis_index("c")
    s = lax.axis_index("s")
    wid = s * _NC + c
    row0 = pl.multiple_of(jnp.minimum(s * _STRIPE, _N - _STRIPE), 8)
    base = wid * _EPW

    zbuf = rows_ring.at[_RB - 1]

    def zvec(i, carry):
        for j in range(_D // 16):
            zbuf[i, pl.ds(j * 16, 16)] = jnp.zeros((16,), jnp.float32)
        return carry

    lax.fori_loop(0, _K, zvec, 0)
    for k in range(_STRIPE // _K):
        pltpu.async_copy(zbuf, acc_sh.at[pl.ds(row0 + k * _K, _K)], zsem)

    def idx_start(chunk):
        bi = chunk % _IB
        off = pl.multiple_of(base + chunk * _K, 8)
        pltpu.async_copy(edges_hbm.at[pl.ds(off, _K)], src_ring.at[bi],
                         isems.at[bi])
        pltpu.async_copy(edges_hbm.at[pl.ds(_E + off, _K)], dst_ring.at[bi],
                         isems.at[bi])

    def idx_wait(chunk):
        bi = chunk % _IB
        off = pl.multiple_of(base + chunk * _K, 8)
        pltpu.make_async_copy(edges_hbm.at[pl.ds(off, _K)], src_ring.at[bi],
                              isems.at[bi]).wait()
        pltpu.make_async_copy(edges_hbm.at[pl.ds(_E + off, _K)],
                              dst_ring.at[bi], isems.at[bi]).wait()

    def gather_start(chunk):
        b = chunk % _RB
        pltpu.async_copy(h_hbm.at[src_ring.at[chunk % _IB]], rows_ring.at[b],
                         gsems.at[b])

    def gather_wait(chunk):
        b = chunk % _RB
        pltpu.make_async_copy(h_hbm.at[src_ring.at[chunk % _IB]],
                              rows_ring.at[b], gsems.at[b]).wait()

    def scatter_add(chunk):
        pltpu.sync_copy(rows_ring.at[chunk % _RB],
                        acc_sh.at[dst_ring.at[chunk % _IB]], add=True)

    for chunk in range(_IB):
        idx_start(chunk)
    for chunk in range(_RB - 1):
        idx_wait(chunk)
        gather_start(chunk)
    for k in range(_STRIPE // _K):
        pltpu.make_async_copy(zbuf, acc_sh.at[pl.ds(row0 + k * _K, _K)],
                              zsem).wait()
    plsc.subcore_barrier()

    def body(chunk, carry):
        gather_wait(chunk)
        scatter_add(chunk)
        ahead = chunk + _RB - 1

        @pl.when(ahead < _CPT)
        def _():
            idx_wait(ahead)
            gather_start(ahead)

        @pl.when(chunk + _IB < _CPT)
        def _():
            idx_start(chunk + _IB)

        return carry

    lax.fori_loop(0, _CPT, body, 0)
    plsc.subcore_barrier()
    pltpu.sync_copy(acc_sh.at[pl.ds(row0, _STRIPE)],
                    out_hbm.at[c].at[pl.ds(row0, _STRIPE)])


_BLK = 1000


def _affine_lrelu_body(acc_ref, w_ref, b_ref, o_ref):
    m = acc_ref[0] + acc_ref[1]
    y = jnp.dot(m, w_ref[...], preferred_element_type=jnp.float32) + b_ref[...]
    o_ref[...] = jnp.where(y >= 0.0, y, 0.1 * y)


def _affine_lrelu(acc, W, b):
    return pl.pallas_call(
        _affine_lrelu_body,
        grid=(_N // _BLK,),
        in_specs=[
            pl.BlockSpec((_NC, _BLK, _D), lambda i: (0, i, 0)),
            pl.BlockSpec((_D, _D), lambda i: (0, 0)),
            pl.BlockSpec((1, _D), lambda i: (0, 0)),
        ],
        out_specs=pl.BlockSpec((_BLK, _D), lambda i: (i, 0)),
        out_shape=jax.ShapeDtypeStruct((_N, _D), jnp.float32),
    )(acc, W, b.reshape(1, _D))


def _partial_body(x_ref, h1_ref, h2_ref, wo_ref, bo_ref, o_ref):
    w = wo_ref[...]
    y = jnp.dot(x_ref[...], w[0:_D], preferred_element_type=jnp.float32)
    y = y + jnp.dot(h1_ref[...], w[_D:2 * _D],
                    preferred_element_type=jnp.float32)
    y = y + jnp.dot(h2_ref[...], w[2 * _D:3 * _D],
                    preferred_element_type=jnp.float32)
    o_ref[...] = y + bo_ref[...]


def _partial_proj(x, h1, h2, W_out3, b_out):
    return pl.pallas_call(
        _partial_body,
        grid=(_N // _BLK,),
        in_specs=[
            pl.BlockSpec((_BLK, _D), lambda i: (i, 0)),
            pl.BlockSpec((_BLK, _D), lambda i: (i, 0)),
            pl.BlockSpec((_BLK, _D), lambda i: (i, 0)),
            pl.BlockSpec((4 * _D, _D), lambda i: (0, 0)),
            pl.BlockSpec((1, _D), lambda i: (0, 0)),
        ],
        out_specs=pl.BlockSpec((_BLK, _D), lambda i: (i, 0)),
        out_shape=jax.ShapeDtypeStruct((_N, _D), jnp.float32),
    )(x, h1, h2, W_out3, b_out.reshape(1, _D))


def _final_body(acc_ref, w2_ref, b2_ref, part_ref, wo3_ref, o_ref):
    m = acc_ref[0] + acc_ref[1]
    y3 = jnp.dot(m, w2_ref[...], preferred_element_type=jnp.float32) + b2_ref[...]
    h3 = jnp.where(y3 >= 0.0, y3, 0.1 * y3)
    y = part_ref[...] + jnp.dot(h3, wo3_ref[3 * _D:4 * _D],
                                preferred_element_type=jnp.float32)
    o_ref[...] = jnp.where(y >= 0.0, y, 0.1 * y)


def _final_proj(acc3, W2, b2, part, W_out_h3):
    return pl.pallas_call(
        _final_body,
        grid=(_N // _BLK,),
        in_specs=[
            pl.BlockSpec((_NC, _BLK, _D), lambda i: (0, i, 0)),
            pl.BlockSpec((_D, _D), lambda i: (0, 0)),
            pl.BlockSpec((1, _D), lambda i: (0, 0)),
            pl.BlockSpec((_BLK, _D), lambda i: (i, 0)),
            pl.BlockSpec((4 * _D, _D), lambda i: (0, 0)),
        ],
        out_specs=pl.BlockSpec((_BLK, _D), lambda i: (i, 0)),
        out_shape=jax.ShapeDtypeStruct((_N, _D), jnp.float32),
    )(acc3, W2, b2.reshape(1, _D), part, W_out_h3)


def kernel(x, edge_index, W0, b0, W1, b1, W2, b2, W_out, b_out):
    edges = edge_index.reshape(2 * _E)
    acc1 = _segment_sum_sc(x, edges)
    h1 = _affine_lrelu(acc1, W0, b0)
    acc2 = _segment_sum_sc(h1, edges)
    h2 = _affine_lrelu(acc2, W1, b1)
    acc3 = _segment_sum_sc(h2, edges)
    part = _partial_proj(x, h1, h2, W_out, b_out)
    return _final_proj(acc3, W2, b2, part, W_out)
